# baseline (speedup 1.0000x reference)
import functools

import jax
import jax.numpy as jnp
from jax import lax
from jax.experimental import pallas as pl
from jax.experimental.pallas import tpu as pltpu
from jax.experimental.pallas import tpu_sc as plsc

VOCAB = 1_000_000
EMBED = 16
BATCH = 16384

NC = 2
NS = 16
L = 16
NW = NC * NS
BPW = BATCH // NW
CHUNK = 128
NCHUNK = BPW // CHUNK
NBLK = BPW // L


def _sc_body(x_hbm, t_hbm, tgt_hbm, ctx_hbm, out_hbm,
             xi_v, ti_v, tgt_rows, ctx_rows, out_v, sem_t, sem_c):
    wid = lax.axis_index("s") * NC + lax.axis_index("c")
    base = wid * BPW

    pltpu.sync_copy(x_hbm.at[pl.ds(wid * NCHUNK, NCHUNK)], xi_v)
    pltpu.sync_copy(t_hbm.at[pl.ds(wid * NCHUNK, NCHUNK)], ti_v)

    copies = []
    for c in range(NCHUNK):
        dst = tgt_rows.at[pl.ds(c * CHUNK, CHUNK)]
        copies.append(pltpu.async_copy(tgt_hbm.at[ti_v.at[c]], dst, sem_t))
    for c in range(NCHUNK):
        dst = ctx_rows.at[pl.ds(c * CHUNK, CHUNK)]
        copies.append(pltpu.async_copy(ctx_hbm.at[xi_v.at[c]], dst, sem_c))
    for cp in copies:
        cp.wait()

    lane = lax.iota(jnp.int32, L)

    def block(j, carry):
        row0 = j * L
        ridx = row0 + lane
        acc = jnp.zeros((L,), jnp.float32)
        for d in range(EMBED):
            cidx = jnp.full((L,), d, jnp.int32)
            tv = plsc.load_gather(tgt_rows, [ridx, cidx])
            cv = plsc.load_gather(ctx_rows, [ridx, cidx])
            acc = acc + tv * cv
        out_v[pl.ds(row0, L)] = acc
        return carry

    lax.fori_loop(0, NBLK, block, 0)

    pltpu.sync_copy(out_v, out_hbm.at[pl.ds(base, BPW)])


@jax.jit
def _skipgram_sc(x2d, t2d, target_table, context_table):
    mesh = plsc.VectorSubcoreMesh(core_axis_name="c", subcore_axis_name="s")
    k = functools.partial(
        pl.kernel,
        mesh=mesh,
        out_type=jax.ShapeDtypeStruct((BATCH,), jnp.float32),
        compiler_params=pltpu.CompilerParams(
            needs_layout_passes=False, use_tc_tiling_on_sc=False),
        scratch_types=[
            pltpu.VMEM((NCHUNK, CHUNK), jnp.int32),
            pltpu.VMEM((NCHUNK, CHUNK), jnp.int32),
            pltpu.VMEM((BPW, EMBED), jnp.float32),
            pltpu.VMEM((BPW, EMBED), jnp.float32),
            pltpu.VMEM((BPW,), jnp.float32),
            pltpu.SemaphoreType.DMA,
            pltpu.SemaphoreType.DMA,
        ],
    )(_sc_body)
    return k(x2d, t2d, target_table, context_table)


def kernel(x, t, target_table, context_table):
    x2d = x.astype(jnp.int32).reshape(BATCH // CHUNK, CHUNK)
    t2d = t.astype(jnp.int32).reshape(BATCH // CHUNK, CHUNK)
    return _skipgram_sc(x2d, t2d, target_table, context_table)


# speedup vs baseline: 5.7761x; 5.7761x over previous
import functools

import jax
import jax.numpy as jnp
from jax import lax
from jax.experimental import pallas as pl
from jax.experimental.pallas import tpu as pltpu
from jax.experimental.pallas import tpu_sc as plsc

VOCAB = 1_000_000
EMBED = 16
BATCH = 16384

NC = 2
NS = 16
L = 16
NW = NC * NS
BPW = BATCH // NW
NGRP = BPW // L
W = 128


def _sc_body(x_hbm, t_hbm, tgt_hbm, ctx_hbm, out_hbm,
             xi_v, ti_v, tgt_stage, ctx_stage, out_v, sem_t, sem_c):
    wid = lax.axis_index("s") * NC + lax.axis_index("c")
    base = wid * BPW

    pltpu.sync_copy(x_hbm.at[pl.ds(wid * 2, 2)], xi_v)
    pltpu.sync_copy(t_hbm.at[pl.ds(wid * 2, 2)], ti_v)

    lane = lax.iota(jnp.int32, L)

    def chunk_body(c, carry):
        r = c // L
        o = (c % L) * L
        vx = xi_v[r, pl.ds(o, L)]
        vt = ti_v[r, pl.ds(o, L)]
        xcol = lane * W + (vx & (W - 1))
        tcol = lane * W + (vt & (W - 1))
        xb = (vx >> 7) * W
        tb = (vt >> 7) * W
        copies = []
        for k in range(L):
            copies.append(pltpu.async_copy(
                tgt_hbm.at[:, :, pl.ds(pl.multiple_of(tb[k], W), W)],
                tgt_stage.at[:, :, pl.ds(k * W, W)], sem_t))
            copies.append(pltpu.async_copy(
                ctx_hbm.at[:, :, pl.ds(pl.multiple_of(xb[k], W), W)],
                ctx_stage.at[:, :, pl.ds(k * W, W)], sem_c))
        for cp in copies:
            cp.wait()

        acc = jnp.zeros((L,), jnp.float32)
        for d in range(EMBED):
            gv = jnp.full((L,), d // 8, jnp.int32)
            sv = jnp.full((L,), d % 8, jnp.int32)
            tv = plsc.load_gather(tgt_stage, [gv, sv, tcol])
            cv = plsc.load_gather(ctx_stage, [gv, sv, xcol])
            acc = acc + tv * cv
        out_v[pl.ds(c * L, L)] = acc
        return carry

    lax.fori_loop(0, NGRP, chunk_body, 0)

    pltpu.sync_copy(out_v, out_hbm.at[pl.ds(base, BPW)])


@jax.jit
def _skipgram_sc(x2d, t2d, tgt3, ctx3):
    mesh = plsc.VectorSubcoreMesh(core_axis_name="c", subcore_axis_name="s")
    k = functools.partial(
        pl.kernel,
        mesh=mesh,
        out_type=jax.ShapeDtypeStruct((BATCH,), jnp.float32),
        compiler_params=pltpu.CompilerParams(
            needs_layout_passes=False,
            use_tc_tiling_on_sc=True),
        scratch_types=[
            pltpu.VMEM((2, 2 * W), jnp.int32),
            pltpu.VMEM((2, 2 * W), jnp.int32),
            pltpu.VMEM((2, 8, L * W), jnp.float32),
            pltpu.VMEM((2, 8, L * W), jnp.float32),
            pltpu.VMEM((BPW,), jnp.float32),
            pltpu.SemaphoreType.DMA,
            pltpu.SemaphoreType.DMA,
        ],
    )(_sc_body)
    return k(x2d, t2d, tgt3, ctx3)


def kernel(x, t, target_table, context_table):
    x2d = x.astype(jnp.int32).reshape(BATCH // (2 * W), 2 * W)
    t2d = t.astype(jnp.int32).reshape(BATCH // (2 * W), 2 * W)
    tgt3 = target_table.T.reshape(2, 8, VOCAB)
    ctx3 = context_table.T.reshape(2, 8, VOCAB)
    return _skipgram_sc(x2d, t2d, tgt3, ctx3)
